# Initial kernel scaffold; baseline (speedup 1.0000x reference)
import jax
import jax.numpy as jnp
from jax.experimental import pallas as pl


def kernel(x, batch, W1, b1, W2, b2):
    raise NotImplementedError("write your pallas kernel here")


# speedup vs baseline: 6.4136x; 6.4136x over previous
import jax
import jax.numpy as jnp
from jax.experimental import pallas as pl
from jax.experimental.pallas import tpu as pltpu

_NUM_SEGMENTS = 64


def _fused_body(x_ref, b_ref, w1_ref, b1_ref, w2_ref, b2_ref, out_ref,
                m_sc, s_sc, acc_sc):
    i = pl.program_id(0)
    nblk = pl.num_programs(0)

    @pl.when(i == 0)
    def _init():
        m_sc[...] = jnp.full_like(m_sc, -jnp.inf)
        s_sc[...] = jnp.zeros_like(s_sc)
        acc_sc[...] = jnp.zeros_like(acc_sc)

    x = x_ref[...]
    seg = b_ref[...][:, 0]
    h = jnp.maximum(
        jnp.dot(x, w1_ref[...], preferred_element_type=jnp.float32)
        + b1_ref[...], 0.0)
    sc = (jnp.dot(h, w2_ref[...], preferred_element_type=jnp.float32)
          + b2_ref[...])[:, 0]

    bsz = x.shape[0]
    seg_ids = jax.lax.broadcasted_iota(jnp.int32, (bsz, _NUM_SEGMENTS), 1)
    onehot = seg[:, None] == seg_ids

    masked = jnp.where(onehot, sc[:, None], -jnp.inf)
    m_blk = jnp.max(masked, axis=0)
    m_old = m_sc[0, :]
    m_new = jnp.maximum(m_old, m_blk)
    alpha = jnp.where(m_new == -jnp.inf, 1.0, jnp.exp(m_old - m_new))

    p = jnp.where(onehot, jnp.exp(sc[:, None] - m_new[None, :]), 0.0)

    m_sc[0, :] = m_new
    s_sc[0, :] = s_sc[0, :] * alpha + jnp.sum(p, axis=0)
    acc_sc[...] = acc_sc[...] * alpha[None, :] + jax.lax.dot_general(
        x, p, (((0,), (0,)), ((), ())), preferred_element_type=jnp.float32)

    @pl.when(i == nblk - 1)
    def _finish():
        s = s_sc[0, :]
        out_ref[...] = jnp.where(
            (s > 0.0)[None, :], acc_sc[...] / s[None, :], 0.0)


def _pick_block(n):
    for b in (2048, 2000, 1600, 1280, 1024, 1000, 800, 640, 512, 500, 400,
              320, 256, 250, 200, 160, 128, 125, 100, 80, 64, 50, 40, 32,
              25, 20, 16, 10, 8, 5, 4, 2, 1):
        if n % b == 0:
            return b
    return 1


def kernel(x, batch, W1, b1, W2, b2):
    n, d = x.shape
    dh = W1.shape[1]
    seg2d = batch.astype(jnp.int32).reshape(n, 1)
    b1r = b1.reshape(1, dh).astype(jnp.float32)
    b2r = b2.reshape(1, 1).astype(jnp.float32)
    bsz = _pick_block(n)
    grid = n // bsz

    return pl.pallas_call(
        _fused_body,
        grid=(grid,),
        in_specs=[
            pl.BlockSpec((bsz, d), lambda i: (i, 0)),
            pl.BlockSpec((bsz, 1), lambda i: (i, 0)),
            pl.BlockSpec((d, dh), lambda i: (0, 0)),
            pl.BlockSpec((1, dh), lambda i: (0, 0)),
            pl.BlockSpec((dh, 1), lambda i: (0, 0)),
            pl.BlockSpec((1, 1), lambda i: (0, 0)),
        ],
        out_specs=pl.BlockSpec((d, _NUM_SEGMENTS), lambda i: (0, 0)),
        out_shape=jax.ShapeDtypeStruct((d, _NUM_SEGMENTS), jnp.float32),
        scratch_shapes=[
            pltpu.VMEM((1, _NUM_SEGMENTS), jnp.float32),
            pltpu.VMEM((1, _NUM_SEGMENTS), jnp.float32),
            pltpu.VMEM((d, _NUM_SEGMENTS), jnp.float32),
        ],
        compiler_params=pltpu.CompilerParams(
            dimension_semantics=("arbitrary",)),
    )(x, seg2d, W1, b1r, W2, b2r).T


# speedup vs baseline: 7.2140x; 1.1248x over previous
import jax
import jax.numpy as jnp
from jax.experimental import pallas as pl
from jax.experimental.pallas import tpu as pltpu

_NUM_SEGMENTS = 64


def _fused_body(x_ref, b_ref, w1_ref, b1_ref, w2_ref, b2_ref, out_ref,
                m_sc, s_sc, acc_sc):
    i = pl.program_id(0)
    nblk = pl.num_programs(0)

    @pl.when(i == 0)
    def _init():
        m_sc[...] = jnp.full_like(m_sc, -jnp.inf)
        s_sc[...] = jnp.zeros_like(s_sc)
        acc_sc[...] = jnp.zeros_like(acc_sc)

    x = x_ref[...]
    seg = b_ref[...][:, 0]
    h = jnp.maximum(
        jnp.dot(x, w1_ref[...], preferred_element_type=jnp.float32)
        + b1_ref[...], 0.0)
    sc = (jnp.dot(h, w2_ref[...], preferred_element_type=jnp.float32)
          + b2_ref[...])[:, 0]

    bsz = x.shape[0]
    seg_ids = jax.lax.broadcasted_iota(jnp.int32, (bsz, _NUM_SEGMENTS), 1)
    onehot = seg[:, None] == seg_ids

    masked = jnp.where(onehot, sc[:, None], -jnp.inf)
    m_blk = jnp.max(masked, axis=0)
    m_old = m_sc[0, :]
    m_new = jnp.maximum(m_old, m_blk)
    alpha = jnp.where(m_new == -jnp.inf, 1.0, jnp.exp(m_old - m_new))

    p = jnp.where(onehot, jnp.exp(sc[:, None] - m_new[None, :]), 0.0)

    m_sc[0, :] = m_new
    s_sc[0, :] = s_sc[0, :] * alpha + jnp.sum(p, axis=0)
    acc_sc[...] = acc_sc[...] * alpha[None, :] + jax.lax.dot_general(
        x, p, (((0,), (0,)), ((), ())), preferred_element_type=jnp.float32)

    @pl.when(i == nblk - 1)
    def _finish():
        s = s_sc[0, :]
        out_ref[...] = jnp.where(
            (s > 0.0)[None, :], acc_sc[...] / s[None, :], 0.0)


def _pick_block(n):
    for b in (10000, 8000, 5000, 4000, 2048, 2000, 1600, 1280, 1024, 1000, 800, 640, 512, 500, 400,
              320, 256, 250, 200, 160, 128, 125, 100, 80, 64, 50, 40, 32,
              25, 20, 16, 10, 8, 5, 4, 2, 1):
        if n % b == 0:
            return b
    return 1


def kernel(x, batch, W1, b1, W2, b2):
    n, d = x.shape
    dh = W1.shape[1]
    seg2d = batch.astype(jnp.int32).reshape(n, 1)
    b1r = b1.reshape(1, dh).astype(jnp.float32)
    b2r = b2.reshape(1, 1).astype(jnp.float32)
    bsz = _pick_block(n)
    grid = n // bsz

    return pl.pallas_call(
        _fused_body,
        grid=(grid,),
        in_specs=[
            pl.BlockSpec((bsz, d), lambda i: (i, 0)),
            pl.BlockSpec((bsz, 1), lambda i: (i, 0)),
            pl.BlockSpec((d, dh), lambda i: (0, 0)),
            pl.BlockSpec((1, dh), lambda i: (0, 0)),
            pl.BlockSpec((dh, 1), lambda i: (0, 0)),
            pl.BlockSpec((1, 1), lambda i: (0, 0)),
        ],
        out_specs=pl.BlockSpec((d, _NUM_SEGMENTS), lambda i: (0, 0)),
        out_shape=jax.ShapeDtypeStruct((d, _NUM_SEGMENTS), jnp.float32),
        scratch_shapes=[
            pltpu.VMEM((1, _NUM_SEGMENTS), jnp.float32),
            pltpu.VMEM((1, _NUM_SEGMENTS), jnp.float32),
            pltpu.VMEM((d, _NUM_SEGMENTS), jnp.float32),
        ],
        compiler_params=pltpu.CompilerParams(
            dimension_semantics=("arbitrary",)),
    )(x, seg2d, W1, b1r, W2, b2r).T
